# Initial kernel scaffold; baseline (speedup 1.0000x reference)
import jax
import jax.numpy as jnp
from jax.experimental import pallas as pl


def kernel(hidden_states, bw1, bb1, ln_g, ln_b, bw2, bb2, tw1, tb1, tw2, tb2):
    raise NotImplementedError("write your pallas kernel here")


# speedup vs baseline: 4.9451x; 4.9451x over previous
import functools
import math

import jax
import jax.numpy as jnp
from jax.experimental import pallas as pl
from jax.experimental.pallas import tpu as pltpu

_SQRT2 = math.sqrt(2.0)
_INTERPRET = False


def _gelu_exact(x):
    return 0.5 * x * (1.0 + jax.lax.erf(x / _SQRT2))


def _mean_body(x_ref, o_ref):
    o_ref[...] = jnp.mean(x_ref[...], axis=1, keepdims=True)


def _router_body(nbt, ne, bs_, max_tok, bwb, take_cap, thr, ms,
                 br_ref, w1_ref, b1_ref, g_ref, be_ref, w2_ref, b2_ref,
                 row_ref, take_ref, slot_ref, idx_ref, act_ref):
    x = br_ref[...]
    h1 = jnp.dot(x, w1_ref[...], preferred_element_type=jnp.float32) + b1_ref[...]
    m = jnp.mean(h1, axis=-1, keepdims=True)
    v = jnp.mean((h1 - m) ** 2, axis=-1, keepdims=True)
    ln = (h1 - m) / jnp.sqrt(v + 1e-5) * g_ref[...] + be_ref[...]
    hg = _gelu_exact(ln)
    logits = jnp.dot(hg, w2_ref[...], preferred_element_type=jnp.float32) + b2_ref[...]
    mx = jnp.max(logits, axis=-1, keepdims=True)
    ex = jnp.exp(logits - mx)
    probs = ex / jnp.sum(ex, axis=-1, keepdims=True)

    p = probs + 1e-10
    ent = -jnp.sum(p * jnp.log(p), axis=-1, keepdims=True) / math.log(ne)
    mask1 = ent > thr

    wv = jnp.max(probs, axis=-1, keepdims=True)
    col = jax.lax.broadcasted_iota(jnp.int32, (nbt, ne), 1)
    ei = jnp.min(jnp.where(probs >= wv, col, ne), axis=-1, keepdims=True)
    ow = jnp.where(col == ei, wv, 0.0)

    total_high = jnp.sum(mask1.astype(jnp.float32))
    riota = jax.lax.broadcasted_iota(jnp.int32, (nbt, 1), 0)
    cur = jnp.where(mask1, ent, -1e30)
    for _ in range(bwb - 1):
        m1 = jnp.max(cur)
        first = jnp.min(jnp.where(cur >= m1, riota, nbt))
        cur = jnp.where(riota == first, -1e30, cur)
    thr_adj = jnp.max(cur)
    adjust = ((total_high * bs_ > max_tok) & (total_high > 0)
              & (total_high > bwb))
    mask2 = jnp.where(adjust, (ent > thr_adj).astype(jnp.float32),
                      mask1.astype(jnp.float32)) > 0.5

    rf = mask2.astype(jnp.float32)
    r0 = jax.lax.broadcasted_iota(jnp.int32, (nbt, nbt), 0)
    r1 = jax.lax.broadcasted_iota(jnp.int32, (nbt, nbt), 1)
    tri = (r1 <= r0).astype(jnp.float32)
    cum = jnp.dot(tri, rf, preferred_element_type=jnp.float32)
    cx = cum - rf

    takeb = mask2 & (cx < take_cap)
    fallb = mask2 & jnp.logical_not(takeb)

    row_ref[...] = jnp.where(fallb | jnp.logical_not(mask1), ow, 0.0)
    take_ref[...] = takeb.astype(jnp.int32)
    slot_ref[...] = jnp.where(takeb, cx, 0.0).astype(jnp.int32)
    riota_f = riota.astype(jnp.float32)
    siota = jax.lax.broadcasted_iota(jnp.int32, (nbt, ms), 1).astype(jnp.float32)
    sel_ms = takeb & (cx == siota)
    idx_ref[...] = jnp.sum(jnp.where(sel_ms, riota_f, 0.0), axis=0,
                           keepdims=True).astype(jnp.int32)
    act_ref[...] = jnp.sum(sel_ms.astype(jnp.float32), axis=0,
                           keepdims=True).astype(jnp.int32)


def _token_body(kti, idx_ref, x_ref, w1_ref, b1_ref, w2_ref, b2_ref, o_ref, acc):
    k = pl.program_id(1)

    @pl.when(k == 0)
    def _():
        acc[...] = jnp.zeros_like(acc)

    acc[...] += jnp.dot(x_ref[0], w1_ref[...], preferred_element_type=jnp.float32)

    @pl.when(k == kti - 1)
    def _():
        h = _gelu_exact(acc[...] + b1_ref[...])
        lo = jnp.dot(h, w2_ref[...], preferred_element_type=jnp.float32) + b2_ref[...]
        mx = jnp.max(lo, axis=-1, keepdims=True)
        ex = jnp.exp(lo - mx)
        o_ref[0] = ex / jnp.sum(ex, axis=-1, keepdims=True)


def _asm_body(nbt, bs_, ne, denom,
              take_ref, slot_ref, rows_ref, tl_ref, o_ref, aux_ref, acc):
    b = pl.program_id(0)

    @pl.when(b == 0)
    def _():
        acc[...] = jnp.zeros_like(acc)

    flag = take_ref[b]
    rowb = rows_ref[pl.ds(b, 1), :]
    out = jnp.where(flag > 0, tl_ref[0], jnp.broadcast_to(rowb, (bs_, ne)))
    o_ref[0] = out
    acc[...] += jnp.sum(out, axis=0, keepdims=True)

    @pl.when(b == nbt - 1)
    def _():
        usage = acc[...] / denom
        t = 1.0 / ne
        aux_ref[...] = jnp.sum(t * jnp.log(t / (usage + 1e-10)),
                               keepdims=True).reshape(1, 1)


def kernel(hidden_states, bw1, bb1, ln_g, ln_b, bw2, bb2, tw1, tb1, tw2, tb2):
    B, S, H = hidden_states.shape
    NE = bw2.shape[1]
    H2 = tw1.shape[1]

    if S <= 4096:
        bs_, thr, budget = 512, 0.6, 0.3
    elif S <= 16384:
        bs_, thr, budget = min(1024, 2048), 0.6 * 1.1, 0.3 * 0.7
    else:
        sf = min(S / 16384, 4)
        bs_, thr, budget = min(int(512 * sf), 2048), 0.6 * 1.2, 0.3 * (1.0 / sf)
    nb = (S + bs_ - 1) // bs_
    padded = nb * bs_
    hs = hidden_states
    if padded > S:
        hs = jnp.concatenate([hs, jnp.zeros((B, padded - S, H), hs.dtype)], axis=1)
    NBT = B * nb
    hs3 = hs.reshape(NBT, bs_, H)

    max_tok = int(S * budget)
    bwb = max(1, max_tok // bs_)
    take_cap = max_tok // bs_
    MS = max(take_cap, 1)

    f32 = jnp.float32
    bb1r = bb1.reshape(1, -1)
    ln_gr = ln_g.reshape(1, -1)
    ln_br = ln_b.reshape(1, -1)
    bb2r = bb2.reshape(1, -1)
    tb1r = tb1.reshape(1, -1)
    tb2r = tb2.reshape(1, -1)

    br = pl.pallas_call(
        _mean_body,
        grid=(NBT,),
        in_specs=[pl.BlockSpec((1, bs_, H), lambda i: (i, 0, 0))],
        out_specs=pl.BlockSpec((1, 1, H), lambda i: (i, 0, 0)),
        out_shape=jax.ShapeDtypeStruct((NBT, 1, H), f32),
        interpret=_INTERPRET,
    )(hs3)
    br = br.reshape(NBT, H)

    rows, take_i, slot_i, idx_i, act_i = pl.pallas_call(
        functools.partial(_router_body, NBT, NE, bs_, max_tok, bwb, take_cap,
                          thr, MS),
        in_specs=[
            pl.BlockSpec((NBT, H), lambda: (0, 0)),
            pl.BlockSpec((H, bw1.shape[1]), lambda: (0, 0)),
            pl.BlockSpec((1, bw1.shape[1]), lambda: (0, 0)),
            pl.BlockSpec((1, bw1.shape[1]), lambda: (0, 0)),
            pl.BlockSpec((1, bw1.shape[1]), lambda: (0, 0)),
            pl.BlockSpec((bw1.shape[1], NE), lambda: (0, 0)),
            pl.BlockSpec((1, NE), lambda: (0, 0)),
        ],
        out_specs=[
            pl.BlockSpec((NBT, NE), lambda: (0, 0)),
            pl.BlockSpec((NBT, 1), lambda: (0, 0)),
            pl.BlockSpec((NBT, 1), lambda: (0, 0)),
            pl.BlockSpec((1, MS), lambda: (0, 0)),
            pl.BlockSpec((1, MS), lambda: (0, 0)),
        ],
        out_shape=[
            jax.ShapeDtypeStruct((NBT, NE), f32),
            jax.ShapeDtypeStruct((NBT, 1), jnp.int32),
            jax.ShapeDtypeStruct((NBT, 1), jnp.int32),
            jax.ShapeDtypeStruct((1, MS), jnp.int32),
            jax.ShapeDtypeStruct((1, MS), jnp.int32),
        ],
        interpret=_INTERPRET,
    )(br, bw1, bb1r, ln_gr, ln_br, bw2, bb2r)

    idx_flat = idx_i.reshape(MS)
    take_flat = take_i.reshape(NBT)
    slot_flat = slot_i.reshape(NBT)

    KT = 1024
    KTI = H // KT
    tl = pl.pallas_call(
        functools.partial(_token_body, KTI),
        grid_spec=pltpu.PrefetchScalarGridSpec(
            num_scalar_prefetch=1,
            grid=(MS, KTI),
            in_specs=[
                pl.BlockSpec((1, bs_, KT), lambda s, k, idx: (idx[s], 0, k)),
                pl.BlockSpec((KT, H2), lambda s, k, idx: (k, 0)),
                pl.BlockSpec((1, H2), lambda s, k, idx: (0, 0)),
                pl.BlockSpec((H2, NE), lambda s, k, idx: (0, 0)),
                pl.BlockSpec((1, NE), lambda s, k, idx: (0, 0)),
            ],
            out_specs=pl.BlockSpec((1, bs_, NE), lambda s, k, idx: (s, 0, 0)),
            scratch_shapes=[pltpu.VMEM((bs_, H2), f32)],
        ),
        out_shape=jax.ShapeDtypeStruct((MS, bs_, NE), f32),
        interpret=_INTERPRET,
    )(idx_flat, hs3, tw1, tb1r, tw2, tb2r)

    rw3, aux_arr = pl.pallas_call(
        functools.partial(_asm_body, NBT, bs_, NE, float(B * S)),
        grid_spec=pltpu.PrefetchScalarGridSpec(
            num_scalar_prefetch=2,
            grid=(NBT,),
            in_specs=[
                pl.BlockSpec((NBT, NE), lambda b, t, sl: (0, 0)),
                pl.BlockSpec((1, bs_, NE), lambda b, t, sl: (sl[b], 0, 0)),
            ],
            out_specs=[
                pl.BlockSpec((1, bs_, NE), lambda b, t, sl: (b, 0, 0)),
                pl.BlockSpec((1, 1), lambda b, t, sl: (0, 0)),
            ],
            scratch_shapes=[pltpu.VMEM((1, NE), f32)],
        ),
        out_shape=[
            jax.ShapeDtypeStruct((NBT, bs_, NE), f32),
            jax.ShapeDtypeStruct((1, 1), f32),
        ],
        interpret=_INTERPRET,
    )(take_flat, slot_flat, rows, tl)

    rw = rw3.reshape(B, padded, NE)[:, :S]
    return rw, aux_arr[0, 0]


# speedup vs baseline: 5.2335x; 1.0583x over previous
import functools
import math

import jax
import jax.numpy as jnp
from jax.experimental import pallas as pl
from jax.experimental.pallas import tpu as pltpu

_SQRT2 = math.sqrt(2.0)
_INTERPRET = False


def _gelu_exact(x):
    return 0.5 * x * (1.0 + jax.lax.erf(x / _SQRT2))


def _mean_body(x_ref, o_ref):
    o_ref[...] = jnp.mean(x_ref[...], axis=1, keepdims=True)


def _router_body(nbt, ne, bs_, max_tok, bwb, take_cap, thr, ms,
                 br_ref, w1_ref, b1_ref, g_ref, be_ref, w2_ref, b2_ref,
                 row_ref, take_ref, slot_ref, idx_ref, act_ref):
    x = br_ref[...]
    h1 = jnp.dot(x, w1_ref[...], preferred_element_type=jnp.float32) + b1_ref[...]
    m = jnp.mean(h1, axis=-1, keepdims=True)
    v = jnp.mean((h1 - m) ** 2, axis=-1, keepdims=True)
    ln = (h1 - m) / jnp.sqrt(v + 1e-5) * g_ref[...] + be_ref[...]
    hg = _gelu_exact(ln)
    logits = jnp.dot(hg, w2_ref[...], preferred_element_type=jnp.float32) + b2_ref[...]
    mx = jnp.max(logits, axis=-1, keepdims=True)
    ex = jnp.exp(logits - mx)
    probs = ex / jnp.sum(ex, axis=-1, keepdims=True)

    p = probs + 1e-10
    ent = -jnp.sum(p * jnp.log(p), axis=-1, keepdims=True) / math.log(ne)
    mask1 = ent > thr

    wv = jnp.max(probs, axis=-1, keepdims=True)
    col = jax.lax.broadcasted_iota(jnp.int32, (nbt, ne), 1)
    ei = jnp.min(jnp.where(probs >= wv, col, ne), axis=-1, keepdims=True)
    ow = jnp.where(col == ei, wv, 0.0)

    total_high = jnp.sum(mask1.astype(jnp.float32))
    riota = jax.lax.broadcasted_iota(jnp.int32, (nbt, 1), 0)
    cur = jnp.where(mask1, ent, -1e30)
    for _ in range(bwb - 1):
        m1 = jnp.max(cur)
        first = jnp.min(jnp.where(cur >= m1, riota, nbt))
        cur = jnp.where(riota == first, -1e30, cur)
    thr_adj = jnp.max(cur)
    adjust = ((total_high * bs_ > max_tok) & (total_high > 0)
              & (total_high > bwb))
    mask2 = jnp.where(adjust, (ent > thr_adj).astype(jnp.float32),
                      mask1.astype(jnp.float32)) > 0.5

    rf = mask2.astype(jnp.float32)
    r0 = jax.lax.broadcasted_iota(jnp.int32, (nbt, nbt), 0)
    r1 = jax.lax.broadcasted_iota(jnp.int32, (nbt, nbt), 1)
    tri = (r1 <= r0).astype(jnp.float32)
    cum = jnp.dot(tri, rf, preferred_element_type=jnp.float32)
    cx = cum - rf

    takeb = mask2 & (cx < take_cap)
    fallb = mask2 & jnp.logical_not(takeb)

    row_ref[...] = jnp.where(fallb | jnp.logical_not(mask1), ow, 0.0)
    take_ref[...] = takeb.astype(jnp.int32)
    slot_ref[...] = jnp.where(takeb, cx, 0.0).astype(jnp.int32)
    riota_f = riota.astype(jnp.float32)
    siota = jax.lax.broadcasted_iota(jnp.int32, (nbt, ms), 1).astype(jnp.float32)
    sel_ms = takeb & (cx == siota)
    idx_ref[...] = jnp.sum(jnp.where(sel_ms, riota_f, 0.0), axis=0,
                           keepdims=True).astype(jnp.int32)
    act_ref[...] = jnp.sum(sel_ms.astype(jnp.float32), axis=0,
                           keepdims=True).astype(jnp.int32)


def _token_body(kti, ms, idx_ref, *refs):
    x_refs = refs[:ms]
    w1_ref, b1_ref, w2_ref, b2_ref, o_ref, acc = refs[ms:]
    k = pl.program_id(0)

    @pl.when(k == 0)
    def _():
        acc[...] = jnp.zeros_like(acc)

    x = jnp.concatenate([r[0] for r in x_refs], axis=0)
    acc[...] += jnp.dot(x, w1_ref[...], preferred_element_type=jnp.float32)

    @pl.when(k == kti - 1)
    def _():
        h = _gelu_exact(acc[...] + b1_ref[...])
        lo = jnp.dot(h, w2_ref[...], preferred_element_type=jnp.float32) + b2_ref[...]
        mx = jnp.max(lo, axis=-1, keepdims=True)
        ex = jnp.exp(lo - mx)
        o_ref[...] = ex / jnp.sum(ex, axis=-1, keepdims=True)


def _asm_body(nbt, bs_, ne, denom,
              take_ref, slot_ref, rows_ref, tl_ref, o_ref, aux_ref, acc):
    b = pl.program_id(0)

    @pl.when(b == 0)
    def _():
        acc[...] = jnp.zeros_like(acc)

    flag = take_ref[b]
    rowb = rows_ref[pl.ds(b, 1), :]
    out = jnp.where(flag > 0, tl_ref[0], jnp.broadcast_to(rowb, (bs_, ne)))
    o_ref[0] = out
    acc[...] += jnp.sum(out, axis=0, keepdims=True)

    @pl.when(b == nbt - 1)
    def _():
        usage = acc[...] / denom
        t = 1.0 / ne
        aux_ref[...] = jnp.sum(t * jnp.log(t / (usage + 1e-10)),
                               keepdims=True).reshape(1, 1)


def kernel(hidden_states, bw1, bb1, ln_g, ln_b, bw2, bb2, tw1, tb1, tw2, tb2):
    B, S, H = hidden_states.shape
    NE = bw2.shape[1]
    H2 = tw1.shape[1]

    if S <= 4096:
        bs_, thr, budget = 512, 0.6, 0.3
    elif S <= 16384:
        bs_, thr, budget = min(1024, 2048), 0.6 * 1.1, 0.3 * 0.7
    else:
        sf = min(S / 16384, 4)
        bs_, thr, budget = min(int(512 * sf), 2048), 0.6 * 1.2, 0.3 * (1.0 / sf)
    nb = (S + bs_ - 1) // bs_
    padded = nb * bs_
    hs = hidden_states
    if padded > S:
        hs = jnp.concatenate([hs, jnp.zeros((B, padded - S, H), hs.dtype)], axis=1)
    NBT = B * nb
    hs3 = hs.reshape(NBT, bs_, H)

    max_tok = int(S * budget)
    bwb = max(1, max_tok // bs_)
    take_cap = max_tok // bs_
    MS = max(take_cap, 1)

    f32 = jnp.float32
    bb1r = bb1.reshape(1, -1)
    ln_gr = ln_g.reshape(1, -1)
    ln_br = ln_b.reshape(1, -1)
    bb2r = bb2.reshape(1, -1)
    tb1r = tb1.reshape(1, -1)
    tb2r = tb2.reshape(1, -1)

    br = pl.pallas_call(
        _mean_body,
        grid=(NBT,),
        in_specs=[pl.BlockSpec((1, bs_, H), lambda i: (i, 0, 0))],
        out_specs=pl.BlockSpec((1, 1, H), lambda i: (i, 0, 0)),
        out_shape=jax.ShapeDtypeStruct((NBT, 1, H), f32),
        interpret=_INTERPRET,
    )(hs3)
    br = br.reshape(NBT, H)

    rows, take_i, slot_i, idx_i, act_i = pl.pallas_call(
        functools.partial(_router_body, NBT, NE, bs_, max_tok, bwb, take_cap,
                          thr, MS),
        in_specs=[
            pl.BlockSpec((NBT, H), lambda: (0, 0)),
            pl.BlockSpec((H, bw1.shape[1]), lambda: (0, 0)),
            pl.BlockSpec((1, bw1.shape[1]), lambda: (0, 0)),
            pl.BlockSpec((1, bw1.shape[1]), lambda: (0, 0)),
            pl.BlockSpec((1, bw1.shape[1]), lambda: (0, 0)),
            pl.BlockSpec((bw1.shape[1], NE), lambda: (0, 0)),
            pl.BlockSpec((1, NE), lambda: (0, 0)),
        ],
        out_specs=[
            pl.BlockSpec((NBT, NE), lambda: (0, 0)),
            pl.BlockSpec((NBT, 1), lambda: (0, 0)),
            pl.BlockSpec((NBT, 1), lambda: (0, 0)),
            pl.BlockSpec((1, MS), lambda: (0, 0)),
            pl.BlockSpec((1, MS), lambda: (0, 0)),
        ],
        out_shape=[
            jax.ShapeDtypeStruct((NBT, NE), f32),
            jax.ShapeDtypeStruct((NBT, 1), jnp.int32),
            jax.ShapeDtypeStruct((NBT, 1), jnp.int32),
            jax.ShapeDtypeStruct((1, MS), jnp.int32),
            jax.ShapeDtypeStruct((1, MS), jnp.int32),
        ],
        interpret=_INTERPRET,
    )(br, bw1, bb1r, ln_gr, ln_br, bw2, bb2r)

    idx_flat = idx_i.reshape(MS)
    take_flat = take_i.reshape(NBT)
    slot_flat = slot_i.reshape(NBT)

    KT = 512
    KTI = H // KT
    x_specs = [
        pl.BlockSpec((1, bs_, KT),
                     functools.partial(lambda s, k, idx: (idx[s], 0, k), s))
        for s in range(MS)
    ]
    tl2 = pl.pallas_call(
        functools.partial(_token_body, KTI, MS),
        grid_spec=pltpu.PrefetchScalarGridSpec(
            num_scalar_prefetch=1,
            grid=(KTI,),
            in_specs=x_specs + [
                pl.BlockSpec((KT, H2), lambda k, idx: (k, 0)),
                pl.BlockSpec((1, H2), lambda k, idx: (0, 0)),
                pl.BlockSpec((H2, NE), lambda k, idx: (0, 0)),
                pl.BlockSpec((1, NE), lambda k, idx: (0, 0)),
            ],
            out_specs=pl.BlockSpec((MS * bs_, NE), lambda k, idx: (0, 0)),
            scratch_shapes=[pltpu.VMEM((MS * bs_, H2), f32)],
        ),
        out_shape=jax.ShapeDtypeStruct((MS * bs_, NE), f32),
        interpret=_INTERPRET,
    )(idx_flat, *([hs3] * MS), tw1, tb1r, tw2, tb2r)
    tl = tl2.reshape(MS, bs_, NE)

    rw3, aux_arr = pl.pallas_call(
        functools.partial(_asm_body, NBT, bs_, NE, float(B * S)),
        grid_spec=pltpu.PrefetchScalarGridSpec(
            num_scalar_prefetch=2,
            grid=(NBT,),
            in_specs=[
                pl.BlockSpec((NBT, NE), lambda b, t, sl: (0, 0)),
                pl.BlockSpec((1, bs_, NE), lambda b, t, sl: (sl[b], 0, 0)),
            ],
            out_specs=[
                pl.BlockSpec((1, bs_, NE), lambda b, t, sl: (b, 0, 0)),
                pl.BlockSpec((1, 1), lambda b, t, sl: (0, 0)),
            ],
            scratch_shapes=[pltpu.VMEM((1, NE), f32)],
        ),
        out_shape=[
            jax.ShapeDtypeStruct((NBT, bs_, NE), f32),
            jax.ShapeDtypeStruct((1, 1), f32),
        ],
        interpret=_INTERPRET,
    )(take_flat, slot_flat, rows, tl)

    rw = rw3.reshape(B, padded, NE)[:, :S]
    return rw, aux_arr[0, 0]


# speedup vs baseline: 5.9979x; 1.1461x over previous
import functools
import math

import jax
import jax.numpy as jnp
from jax.experimental import pallas as pl
from jax.experimental.pallas import tpu as pltpu

_SQRT2 = math.sqrt(2.0)
_INTERPRET = False


def _gelu_exact(x):
    return 0.5 * x * (1.0 + jax.lax.erf(x / _SQRT2))


def _router_math(brs, w1_ref, b1_ref, g_ref, be_ref, w2_ref, b2_ref,
                 nbt, ne, bs_, max_tok, bwb, take_cap, thr, ms):
    h1 = jnp.dot(brs, w1_ref[...], preferred_element_type=jnp.float32) + b1_ref[...]
    m = jnp.mean(h1, axis=-1, keepdims=True)
    v = jnp.mean((h1 - m) ** 2, axis=-1, keepdims=True)
    ln = (h1 - m) / jnp.sqrt(v + 1e-5) * g_ref[...] + be_ref[...]
    hg = _gelu_exact(ln)
    logits = jnp.dot(hg, w2_ref[...], preferred_element_type=jnp.float32) + b2_ref[...]
    mx = jnp.max(logits, axis=-1, keepdims=True)
    ex = jnp.exp(logits - mx)
    probs = ex / jnp.sum(ex, axis=-1, keepdims=True)

    p = probs + 1e-10
    ent = -jnp.sum(p * jnp.log(p), axis=-1, keepdims=True) / math.log(ne)
    mask1 = ent > thr

    wv = jnp.max(probs, axis=-1, keepdims=True)
    col = jax.lax.broadcasted_iota(jnp.int32, (nbt, ne), 1)
    ei = jnp.min(jnp.where(probs >= wv, col, ne), axis=-1, keepdims=True)
    ow = jnp.where(col == ei, wv, 0.0)

    total_high = jnp.sum(mask1.astype(jnp.float32))
    riota = jax.lax.broadcasted_iota(jnp.int32, (nbt, 1), 0)
    cur = jnp.where(mask1, ent, -1e30)
    for _ in range(bwb - 1):
        m1 = jnp.max(cur)
        first = jnp.min(jnp.where(cur >= m1, riota, nbt))
        cur = jnp.where(riota == first, -1e30, cur)
    thr_adj = jnp.max(cur)
    adjust = ((total_high * bs_ > max_tok) & (total_high > 0)
              & (total_high > bwb))
    mask2 = jnp.where(adjust, (ent > thr_adj).astype(jnp.float32),
                      mask1.astype(jnp.float32)) > 0.5

    rf = mask2.astype(jnp.float32)
    r0 = jax.lax.broadcasted_iota(jnp.int32, (nbt, nbt), 0)
    r1 = jax.lax.broadcasted_iota(jnp.int32, (nbt, nbt), 1)
    tri = (r1 <= r0).astype(jnp.float32)
    cum = jnp.dot(tri, rf, preferred_element_type=jnp.float32)
    cx = cum - rf

    takeb = mask2 & (cx < take_cap)
    fallb = mask2 & jnp.logical_not(takeb)

    rows = jnp.where(fallb | jnp.logical_not(mask1), ow, 0.0)
    riota_f = riota.astype(jnp.float32)
    siota = jax.lax.broadcasted_iota(jnp.int32, (nbt, ms), 1).astype(jnp.float32)
    sel_ms = takeb & (cx == siota)
    idx = jnp.sum(jnp.where(sel_ms, riota_f, 0.0), axis=0,
                  keepdims=True).astype(jnp.int32)
    act = jnp.sum(sel_ms.astype(jnp.float32), axis=0,
                  keepdims=True).astype(jnp.int32)
    return rows, idx, act


def _stage_a_body(nbt, ne, bs_, max_tok, bwb, take_cap, thr, ms,
                  x_ref, w1_ref, b1_ref, g_ref, be_ref, w2_ref, b2_ref,
                  rows_ref, idx_ref, act_ref, brs):
    i = pl.program_id(0)
    brs[pl.ds(i, 1), :] = jnp.mean(x_ref[0], axis=0, keepdims=True)

    @pl.when(i == nbt - 1)
    def _():
        rows, idx, act = _router_math(
            brs[...], w1_ref, b1_ref, g_ref, be_ref, w2_ref, b2_ref,
            nbt, ne, bs_, max_tok, bwb, take_cap, thr, ms)
        rows_ref[...] = rows
        idx_ref[...] = idx
        act_ref[...] = act


def _stage_b_body(kti, ms, nbt, bs_, ne, denom, idx_ref, act_ref, *refs):
    x_refs = refs[:ms]
    w1_ref, b1_ref, w2_ref, b2_ref, rows_ref, rw_ref, aux_ref, acc = refs[ms:]
    k = pl.program_id(0)

    @pl.when(k == 0)
    def _():
        acc[...] = jnp.zeros_like(acc)

    x = jnp.concatenate([r[0] for r in x_refs], axis=0)
    acc[...] += jnp.dot(x, w1_ref[...], preferred_element_type=jnp.float32)

    @pl.when(k == kti - 1)
    def _():
        h = _gelu_exact(acc[...] + b1_ref[...])
        lo = jnp.dot(h, w2_ref[...], preferred_element_type=jnp.float32) + b2_ref[...]
        mx = jnp.max(lo, axis=-1, keepdims=True)
        exl = jnp.exp(lo - mx)
        tl = exl / jnp.sum(exl, axis=-1, keepdims=True)

        rows = rows_ref[...]
        rw_ref[...] = jnp.broadcast_to(rows[:, None, :], (nbt, bs_, ne))
        usage = jnp.sum(rows, axis=0, keepdims=True) * bs_
        for s in range(ms):
            tls = tl[s * bs_:(s + 1) * bs_]

            @pl.when(act_ref[s] > 0)
            def _(tls=tls, s=s):
                rw_ref[pl.ds(idx_ref[s], 1)] = tls[None]

            usage = usage + jnp.where(
                act_ref[s] > 0,
                jnp.sum(tls, axis=0, keepdims=True),
                jnp.zeros((1, ne), jnp.float32))
        usage = usage / denom
        t = 1.0 / ne
        aux_ref[...] = jnp.sum(t * jnp.log(t / (usage + 1e-10)),
                               keepdims=True).reshape(1, 1)


def kernel(hidden_states, bw1, bb1, ln_g, ln_b, bw2, bb2, tw1, tb1, tw2, tb2):
    B, S, H = hidden_states.shape
    NE = bw2.shape[1]
    BRD = bw1.shape[1]
    H2 = tw1.shape[1]

    if S <= 4096:
        bs_, thr, budget = 512, 0.6, 0.3
    elif S <= 16384:
        bs_, thr, budget = min(1024, 2048), 0.6 * 1.1, 0.3 * 0.7
    else:
        sf = min(S / 16384, 4)
        bs_, thr, budget = min(int(512 * sf), 2048), 0.6 * 1.2, 0.3 * (1.0 / sf)
    nb = (S + bs_ - 1) // bs_
    padded = nb * bs_
    hs = hidden_states
    if padded > S:
        hs = jnp.concatenate([hs, jnp.zeros((B, padded - S, H), hs.dtype)], axis=1)
    NBT = B * nb
    hs3 = hs.reshape(NBT, bs_, H)

    max_tok = int(S * budget)
    bwb = max(1, max_tok // bs_)
    take_cap = max_tok // bs_
    MS = max(take_cap, 1)

    f32 = jnp.float32
    bb1r = bb1.reshape(1, -1)
    ln_gr = ln_g.reshape(1, -1)
    ln_br = ln_b.reshape(1, -1)
    bb2r = bb2.reshape(1, -1)
    tb1r = tb1.reshape(1, -1)
    tb2r = tb2.reshape(1, -1)

    rows, idx_i, act_i = pl.pallas_call(
        functools.partial(_stage_a_body, NBT, NE, bs_, max_tok, bwb, take_cap,
                          thr, MS),
        grid=(NBT,),
        in_specs=[
            pl.BlockSpec((1, bs_, H), lambda i: (i, 0, 0)),
            pl.BlockSpec((H, BRD), lambda i: (0, 0)),
            pl.BlockSpec((1, BRD), lambda i: (0, 0)),
            pl.BlockSpec((1, BRD), lambda i: (0, 0)),
            pl.BlockSpec((1, BRD), lambda i: (0, 0)),
            pl.BlockSpec((BRD, NE), lambda i: (0, 0)),
            pl.BlockSpec((1, NE), lambda i: (0, 0)),
        ],
        out_specs=[
            pl.BlockSpec((NBT, NE), lambda i: (0, 0)),
            pl.BlockSpec((1, MS), lambda i: (0, 0)),
            pl.BlockSpec((1, MS), lambda i: (0, 0)),
        ],
        out_shape=[
            jax.ShapeDtypeStruct((NBT, NE), f32),
            jax.ShapeDtypeStruct((1, MS), jnp.int32),
            jax.ShapeDtypeStruct((1, MS), jnp.int32),
        ],
        scratch_shapes=[pltpu.VMEM((NBT, H), f32)],
        interpret=_INTERPRET,
    )(hs3, bw1, bb1r, ln_gr, ln_br, bw2, bb2r)

    idx_flat = idx_i.reshape(MS)
    act_flat = act_i.reshape(MS)

    KT = 512
    KTI = H // KT
    x_specs = [
        pl.BlockSpec((1, bs_, KT),
                     functools.partial(lambda s, k, idx, act: (idx[s], 0, k), s))
        for s in range(MS)
    ]
    rw3, aux_arr = pl.pallas_call(
        functools.partial(_stage_b_body, KTI, MS, NBT, bs_, NE, float(B * S)),
        grid_spec=pltpu.PrefetchScalarGridSpec(
            num_scalar_prefetch=2,
            grid=(KTI,),
            in_specs=x_specs + [
                pl.BlockSpec((KT, H2), lambda k, idx, act: (k, 0)),
                pl.BlockSpec((1, H2), lambda k, idx, act: (0, 0)),
                pl.BlockSpec((H2, NE), lambda k, idx, act: (0, 0)),
                pl.BlockSpec((1, NE), lambda k, idx, act: (0, 0)),
                pl.BlockSpec((NBT, NE), lambda k, idx, act: (0, 0)),
            ],
            out_specs=[
                pl.BlockSpec((NBT, bs_, NE), lambda k, idx, act: (0, 0, 0)),
                pl.BlockSpec((1, 1), lambda k, idx, act: (0, 0)),
            ],
            scratch_shapes=[pltpu.VMEM((MS * bs_, H2), f32)],
        ),
        out_shape=[
            jax.ShapeDtypeStruct((NBT, bs_, NE), f32),
            jax.ShapeDtypeStruct((1, 1), f32),
        ],
        interpret=_INTERPRET,
    )(idx_flat, act_flat, *([hs3] * MS), tw1, tb1r, tw2, tb2r, rows)

    rw = rw3.reshape(B, padded, NE)[:, :S]
    return rw, aux_arr[0, 0]
